# baseline (speedup 1.0000x reference)
import functools

import jax
import jax.numpy as jnp
from jax import lax
from jax.experimental import pallas as pl
from jax.experimental.pallas import tpu as pltpu
from jax.experimental.pallas import tpu_sc as plsc

F32 = jnp.float32
NB = 1000


def _prep_body(x_ref, batch_ref, table_ref, b_ref, h0_ref):
    nb = x_ref.shape[0]
    iota_g = lax.broadcasted_iota(jnp.int32, (nb, 128), 1)
    b_ref[...] = (iota_g == batch_ref[...]).astype(F32)
    onehot_x = (iota_g == x_ref[...]).astype(F32)
    h0_ref[...] = jnp.dot(onehot_x, table_ref[...], preferred_element_type=F32)


def _prep(x2, batch2, table_pad, n):
    grid = (n // NB,)
    return pl.pallas_call(
        _prep_body,
        grid=grid,
        in_specs=[
            pl.BlockSpec((NB, 1), lambda i: (i, 0)),
            pl.BlockSpec((NB, 1), lambda i: (i, 0)),
            pl.BlockSpec(table_pad.shape, lambda i: (0, 0)),
        ],
        out_specs=[
            pl.BlockSpec((NB, 128), lambda i: (i, 0)),
            pl.BlockSpec((NB, table_pad.shape[1]), lambda i: (i, 0)),
        ],
        out_shape=[
            jax.ShapeDtypeStruct((n, 128), F32),
            jax.ShapeDtypeStruct((n, table_pad.shape[1]), F32),
        ],
    )(x2, batch2, table_pad)


def _write_y(y_ref, hl, bond_ref):
    for j in range(10):
        h, a = divmod(j, 5)
        sl = slice(h * 128, (h + 1) * 128)
        y_ref[j] = jnp.maximum(hl[:, sl] + bond_ref[a:a + 1, sl], 0.0)


def _ybuild_body(h_ref, bond_ref, y_ref):
    _write_y(y_ref, h_ref[...], bond_ref)


def _ybuild(h0, bond, n, emb):
    grid = (n // NB,)
    return pl.pallas_call(
        _ybuild_body,
        grid=grid,
        in_specs=[
            pl.BlockSpec((NB, emb), lambda i: (i, 0)),
            pl.BlockSpec((5, emb), lambda i: (0, 0)),
        ],
        out_specs=pl.BlockSpec((10, NB, 128), lambda i: (0, i, 0)),
        out_shape=jax.ShapeDtypeStruct((10, n, 128), F32),
    )(h0, bond)


def _t1b_body(z2_ref, ss_ref, hprev_ref, b_ref, vne_ref, bond_ref,
              hl_ref, y_ref):
    hnew = jnp.maximum(z2_ref[...] * ss_ref[0:1, :] + ss_ref[1:2, :], 0.0)
    hnew = hnew + hprev_ref[...]
    hl = hnew + jnp.dot(b_ref[...], vne_ref[...], preferred_element_type=F32)
    hl_ref[...] = hl
    _write_y(y_ref, hl, bond_ref)


def _t1b(z2, ss2, hprev, b, vne, bond, n, emb):
    grid = (n // NB,)
    return pl.pallas_call(
        _t1b_body,
        grid=grid,
        in_specs=[
            pl.BlockSpec((NB, emb), lambda i: (i, 0)),
            pl.BlockSpec((2, emb), lambda i: (0, 0)),
            pl.BlockSpec((NB, emb), lambda i: (i, 0)),
            pl.BlockSpec((NB, 128), lambda i: (i, 0)),
            pl.BlockSpec((128, emb), lambda i: (0, 0)),
            pl.BlockSpec((5, emb), lambda i: (0, 0)),
        ],
        out_specs=[
            pl.BlockSpec((NB, emb), lambda i: (i, 0)),
            pl.BlockSpec((10, NB, 128), lambda i: (0, i, 0)),
        ],
        out_shape=[
            jax.ShapeDtypeStruct((n, emb), F32),
            jax.ShapeDtypeStruct((10, n, 128), F32),
        ],
    )(z2, ss2, hprev, b, vne, bond)


def _mlp_stats_tail(i, nblk, n_rows, z, g_ref, bz_ref, ss_ref, acc_s, acc_q):
    s = jnp.sum(z, axis=0, keepdims=True)
    q = jnp.sum(z * z, axis=0, keepdims=True)

    @pl.when(i == 0)
    def _():
        acc_s[...] = s
        acc_q[...] = q

    @pl.when(i > 0)
    def _():
        acc_s[...] += s
        acc_q[...] += q

    @pl.when(i == nblk - 1)
    def _():
        mean = acc_s[...] * (1.0 / n_rows)
        var = acc_q[...] * (1.0 / n_rows) - mean * mean
        scale = lax.rsqrt(var + 1e-5) * g_ref[...]
        ss_ref[0:1, :] = scale
        ss_ref[1:2, :] = bz_ref[...] - mean * scale


def _p1_body(n_rows, hl_ref, agg_ref, eps_ref, w1_ref, g_ref, bz_ref,
             z1_ref, ss_ref, acc_s, acc_q):
    i = pl.program_id(0)
    nblk = pl.num_programs(0)
    aggr = jnp.concatenate([agg_ref[0], agg_ref[1]], axis=1)
    z = hl_ref[...] * eps_ref[...] + aggr
    z1 = jnp.dot(z, w1_ref[...], preferred_element_type=F32)
    z1_ref[...] = z1
    _mlp_stats_tail(i, nblk, n_rows, z1, g_ref, bz_ref, ss_ref, acc_s, acc_q)


def _p1(hl, aggr3, eps_row, w1, g1, bz1, n, emb):
    grid = (n // NB,)
    h = 2 * emb
    return pl.pallas_call(
        functools.partial(_p1_body, float(n)),
        grid=grid,
        in_specs=[
            pl.BlockSpec((NB, emb), lambda i: (i, 0)),
            pl.BlockSpec((2, NB, 128), lambda i: (0, i, 0)),
            pl.BlockSpec((1, emb), lambda i: (0, 0)),
            pl.BlockSpec((emb, h), lambda i: (0, 0)),
            pl.BlockSpec((1, h), lambda i: (0, 0)),
            pl.BlockSpec((1, h), lambda i: (0, 0)),
        ],
        out_specs=[
            pl.BlockSpec((NB, h), lambda i: (i, 0)),
            pl.BlockSpec((2, h), lambda i: (0, 0)),
        ],
        out_shape=[
            jax.ShapeDtypeStruct((n, h), F32),
            jax.ShapeDtypeStruct((2, h), F32),
        ],
        scratch_shapes=[pltpu.VMEM((1, h), F32), pltpu.VMEM((1, h), F32)],
    )(hl, aggr3, eps_row, w1, g1, bz1)


def _p2_body(n_rows, z1_ref, ss1_ref, w2_ref, g_ref, bz_ref,
             z2_ref, ss2_ref, acc_s, acc_q):
    i = pl.program_id(0)
    nblk = pl.num_programs(0)
    a = jnp.maximum(z1_ref[...] * ss1_ref[0:1, :] + ss1_ref[1:2, :], 0.0)
    z2 = jnp.dot(a, w2_ref[...], preferred_element_type=F32)
    z2_ref[...] = z2
    _mlp_stats_tail(i, nblk, n_rows, z2, g_ref, bz_ref, ss2_ref, acc_s, acc_q)


def _p2(z1, ss1, w2, g2, bz2, n, emb):
    grid = (n // NB,)
    h = 2 * emb
    return pl.pallas_call(
        functools.partial(_p2_body, float(n)),
        grid=grid,
        in_specs=[
            pl.BlockSpec((NB, h), lambda i: (i, 0)),
            pl.BlockSpec((2, h), lambda i: (0, 0)),
            pl.BlockSpec((h, emb), lambda i: (0, 0)),
            pl.BlockSpec((1, emb), lambda i: (0, 0)),
            pl.BlockSpec((1, emb), lambda i: (0, 0)),
        ],
        out_specs=[
            pl.BlockSpec((NB, emb), lambda i: (i, 0)),
            pl.BlockSpec((2, emb), lambda i: (0, 0)),
        ],
        out_shape=[
            jax.ShapeDtypeStruct((n, emb), F32),
            jax.ShapeDtypeStruct((2, emb), F32),
        ],
        scratch_shapes=[pltpu.VMEM((1, emb), F32), pltpu.VMEM((1, emb), F32)],
    )(z1, ss1, w2, g2, bz2)


def _bn_rows(t, g, b):
    mean = jnp.mean(t, axis=0, keepdims=True)
    var = jnp.mean(jnp.square(t - mean), axis=0, keepdims=True)
    return (t - mean) * lax.rsqrt(var + 1e-5) * g + b


def _vn_body(hl_ref, b_ref, vne_ref, wa_ref, ga_ref, bza_ref,
             wb_ref, gb_ref, bzb_ref, vout_ref, acc):
    i = pl.program_id(0)
    nblk = pl.num_programs(0)
    part = lax.dot_general(b_ref[...], hl_ref[...],
                           (((0,), (0,)), ((), ())),
                           preferred_element_type=F32)

    @pl.when(i == 0)
    def _():
        acc[...] = part

    @pl.when(i > 0)
    def _():
        acc[...] += part

    @pl.when(i == nblk - 1)
    def _():
        tmp = acc[...] + vne_ref[...]
        t = jnp.dot(tmp, wa_ref[...], preferred_element_type=F32)
        t = jnp.maximum(_bn_rows(t, ga_ref[...], bza_ref[...]), 0.0)
        t = jnp.dot(t, wb_ref[...], preferred_element_type=F32)
        t = jnp.maximum(_bn_rows(t, gb_ref[...], bzb_ref[...]), 0.0)
        vout_ref[...] = vne_ref[...] + t


def _vn(hl, b, vne, wa, ga, bza, wb, gb, bzb, n, emb):
    grid = (n // NB,)
    h = 2 * emb
    return pl.pallas_call(
        _vn_body,
        grid=grid,
        in_specs=[
            pl.BlockSpec((NB, emb), lambda i: (i, 0)),
            pl.BlockSpec((NB, 128), lambda i: (i, 0)),
            pl.BlockSpec((128, emb), lambda i: (0, 0)),
            pl.BlockSpec((emb, h), lambda i: (0, 0)),
            pl.BlockSpec((1, h), lambda i: (0, 0)),
            pl.BlockSpec((1, h), lambda i: (0, 0)),
            pl.BlockSpec((h, emb), lambda i: (0, 0)),
            pl.BlockSpec((1, emb), lambda i: (0, 0)),
            pl.BlockSpec((1, emb), lambda i: (0, 0)),
        ],
        out_specs=pl.BlockSpec((128, emb), lambda i: (0, 0)),
        out_shape=jax.ShapeDtypeStruct((128, emb), F32),
        scratch_shapes=[pltpu.VMEM((128, emb), F32)],
    )(hl, b, vne, wa, ga, bza, wb, gb, bzb)


def _final_body(z2_ref, ss2_ref, hl_ref, b_ref, pw_ref, pb_ref,
                out_ref, acc, cnt):
    i = pl.program_id(0)
    nblk = pl.num_programs(0)
    hnode = z2_ref[...] * ss2_ref[0:1, :] + ss2_ref[1:2, :] + hl_ref[...]
    bblk = b_ref[...]
    part = lax.dot_general(bblk, hnode, (((0,), (0,)), ((), ())),
                           preferred_element_type=F32)
    ones = jnp.ones(bblk.shape[:1] + (128,), F32)
    c = lax.dot_general(bblk, ones, (((0,), (0,)), ((), ())),
                        preferred_element_type=F32)

    @pl.when(i == 0)
    def _():
        acc[...] = part
        cnt[...] = c

    @pl.when(i > 0)
    def _():
        acc[...] += part
        cnt[...] += c

    @pl.when(i == nblk - 1)
    def _():
        hg = acc[...] / jnp.maximum(cnt[:, 0:1], 1.0)
        out_ref[...] = (jnp.dot(hg, pw_ref[...], preferred_element_type=F32)
                        + pb_ref[...])


def _final(z2, ss2, hl, b, pw, pb, n, emb, ncls):
    grid = (n // NB,)
    return pl.pallas_call(
        _final_body,
        grid=grid,
        in_specs=[
            pl.BlockSpec((NB, emb), lambda i: (i, 0)),
            pl.BlockSpec((2, emb), lambda i: (0, 0)),
            pl.BlockSpec((NB, emb), lambda i: (i, 0)),
            pl.BlockSpec((NB, 128), lambda i: (i, 0)),
            pl.BlockSpec((emb, ncls), lambda i: (0, 0)),
            pl.BlockSpec((1, ncls), lambda i: (0, 0)),
        ],
        out_specs=pl.BlockSpec((128, ncls), lambda i: (0, 0)),
        out_shape=jax.ShapeDtypeStruct((128, ncls), F32),
        scratch_shapes=[pltpu.VMEM((128, emb), F32), pltpu.VMEM((128, 128), F32)],
    )(z2, ss2, hl, b, pw, pb)


def _edge_aggr(y_flat, keys5, dst4, zeros, npad):
    nq = keys5.shape[2]
    chunk = 128
    nchunks = nq * 4
    assert nq % 2 == 0
    rp = npad // 16
    assert rp % 8 == 0
    mesh = plsc.VectorSubcoreMesh(core_axis_name="c", subcore_axis_name="s",
                                  num_cores=2, num_subcores=16)

    @functools.partial(
        pl.kernel, mesh=mesh,
        out_type=jax.ShapeDtypeStruct((2 * npad, 128), F32),
        scratch_types=[
            pltpu.VMEM((4, chunk), jnp.int32),
            pltpu.VMEM((4, chunk), jnp.int32),
            pltpu.VMEM((4, chunk), jnp.int32),
            pltpu.VMEM((4, chunk), jnp.int32),
            pltpu.VMEM((chunk, 128), F32),
            pltpu.VMEM((chunk, 128), F32),
            pltpu.VMEM_SHARED((npad, 128), F32),
            pltpu.SemaphoreType.DMA,
            pltpu.SemaphoreType.DMA,
            pltpu.SemaphoreType.DMA,
        ],
    )
    def k(y_hbm, keys_hbm, dst_hbm, zeros_hbm, out_hbm,
          kqa, kqb, dqa, dqb, ra, rb, acc, sa, sb, sz):
        kq = [kqa, kqb]
        dq = [dqa, dqb]
        rows = [ra, rb]
        sems = [sa, sb]
        c = lax.axis_index("c")
        s = lax.axis_index("s")

        pltpu.async_copy(zeros_hbm.at[pl.ds(s * rp, rp)],
                         acc.at[pl.ds(s * rp, rp)], sz)

        def load_quad(q, b):
            pltpu.sync_copy(keys_hbm.at[c, s, q], kq[b])
            pltpu.sync_copy(dst_hbm.at[s, q], dq[b])

        load_quad(0, 0)
        pltpu.make_async_copy(zeros_hbm.at[pl.ds(s * rp, rp)],
                              acc.at[pl.ds(s * rp, rp)], sz).wait()
        plsc.subcore_barrier()
        pltpu.async_copy(y_hbm.at[kq[0].at[0]], rows[0], sems[0])

        niter = nq // 2

        def body(i, carry):
            k0 = 8 * i
            load_quad(2 * i + 1, 1)
            for j in range(8):
                kk = k0 + j
                qb = j // 4
                row = j % 4
                rpar = j % 2
                if j < 7:
                    nqb = (j + 1) // 4
                    nrow = (j + 1) % 4

                    @pl.when(kk + 1 < nchunks)
                    def _():
                        pltpu.async_copy(y_hbm.at[kq[nqb].at[nrow]],
                                         rows[1 - rpar], sems[1 - rpar])
                if j == 6:
                    @pl.when(i + 1 < niter)
                    def _():
                        load_quad(2 * i + 2, 0)
                if j == 7:
                    @pl.when(i + 1 < niter)
                    def _():
                        pltpu.async_copy(y_hbm.at[kq[0].at[0]],
                                         rows[0], sems[0])
                pltpu.make_async_copy(y_hbm.at[kq[qb].at[row]],
                                      rows[rpar], sems[rpar]).wait()
                pltpu.sync_copy(rows[rpar], acc.at[dq[qb].at[row]], add=True)
            return carry

        lax.fori_loop(0, niter, body, 0)
        plsc.subcore_barrier()
        pltpu.sync_copy(acc.at[pl.ds(s * rp, rp)],
                        out_hbm.at[pl.ds(c * npad + s * rp, rp)])

    return k(y_flat, keys5, dst4, zeros)


def kernel(x, edge_index, edge_attr, batch, params):
    n = x.shape[0]
    e = edge_attr.shape[0]
    emb = params["atom_table"].shape[1]
    ncls = params["pred_W"].shape[1]
    nlayer = len(params["convs"])
    assert n % NB == 0 and emb == 256 and e % 16 == 0

    npad = ((n // 16 + 127) // 128) * 128 * 16

    x2 = x.astype(jnp.int32).reshape(n, 1)
    batch2 = batch.astype(jnp.int32).reshape(n, 1)
    vocab = params["atom_table"].shape[0]
    table_pad = jnp.concatenate(
        [params["atom_table"], jnp.zeros((128 - vocab, emb), F32)], axis=0)

    src = edge_index[0].astype(jnp.int32)
    dst = edge_index[1].astype(jnp.int32)
    ep = e // 16
    quad = 512
    nqd = -(-ep // (2 * quad)) * 2
    epp = nqd * quad
    keys = edge_attr.astype(jnp.int32) * n + src
    keys_t = jnp.pad(keys.reshape(16, ep), ((0, 0), (0, epp - ep)))
    keys5 = jnp.stack([keys_t, keys_t + 5 * n]).reshape(2, 16, nqd, 4, 128)
    dst4 = jnp.pad(dst.reshape(16, ep), ((0, 0), (0, epp - ep)),
                   constant_values=n).reshape(16, nqd, 4, 128)
    zeros = jnp.zeros((npad, 128), F32)

    b_onehot, h0 = _prep(x2, batch2, table_pad, n)

    hl = h0
    z2 = ss2 = None
    vne = None
    for l in range(nlayer):
        p = params["convs"][l]
        bond = p["bond_table"]
        if l == 0:
            y = _ybuild(hl, bond, n, emb)
        else:
            hl, y = _t1b(z2, ss2, hl, b_onehot, vne, bond, n, emb)
        aggr = _edge_aggr(y.reshape(10 * n, 128), keys5, dst4, zeros, npad)
        eps_row = (1.0 + p["eps"]) * jnp.ones((1, emb), F32)
        z1, ss1 = _p1(hl, aggr.reshape(2, npad, 128), eps_row, p["W1"],
                      p["bn1_g"].reshape(1, -1), p["bn1_b"].reshape(1, -1),
                      n, emb)
        z2, ss2 = _p2(z1, ss1, p["W2"],
                      p["bn_g"].reshape(1, -1), p["bn_b"].reshape(1, -1),
                      n, emb)
        if l < nlayer - 1:
            mp = params["vn_mlps"][l]
            vne_in = vne if vne is not None else jnp.zeros((128, emb), F32)
            vne = _vn(hl, b_onehot, vne_in, mp["Wa"],
                      mp["bn_a_g"].reshape(1, -1), mp["bn_a_b"].reshape(1, -1),
                      mp["Wb"],
                      mp["bn_b_g"].reshape(1, -1), mp["bn_b_b"].reshape(1, -1),
                      n, emb)

    return _final(z2, ss2, hl, b_onehot, params["pred_W"],
                  params["pred_b"].reshape(1, -1), n, emb, ncls)


# speedup vs baseline: 1.0153x; 1.0153x over previous
import functools

import jax
import jax.numpy as jnp
from jax import lax
from jax.experimental import pallas as pl
from jax.experimental.pallas import tpu as pltpu
from jax.experimental.pallas import tpu_sc as plsc

F32 = jnp.float32
NB = 1000


def _prep_body(x_ref, batch_ref, table_ref, b_ref, h0_ref):
    nb = x_ref.shape[0]
    iota_g = lax.broadcasted_iota(jnp.int32, (nb, 128), 1)
    b_ref[...] = (iota_g == batch_ref[...]).astype(F32)
    onehot_x = (iota_g == x_ref[...]).astype(F32)
    h0_ref[...] = jnp.dot(onehot_x, table_ref[...], preferred_element_type=F32)


def _prep(x2, batch2, table_pad, n):
    grid = (n // NB,)
    return pl.pallas_call(
        _prep_body,
        grid=grid,
        in_specs=[
            pl.BlockSpec((NB, 1), lambda i: (i, 0)),
            pl.BlockSpec((NB, 1), lambda i: (i, 0)),
            pl.BlockSpec(table_pad.shape, lambda i: (0, 0)),
        ],
        out_specs=[
            pl.BlockSpec((NB, 128), lambda i: (i, 0)),
            pl.BlockSpec((NB, table_pad.shape[1]), lambda i: (i, 0)),
        ],
        out_shape=[
            jax.ShapeDtypeStruct((n, 128), F32),
            jax.ShapeDtypeStruct((n, table_pad.shape[1]), F32),
        ],
    )(x2, batch2, table_pad)


def _write_y(y_ref, hl, bond_ref):
    for j in range(10):
        h, a = divmod(j, 5)
        sl = slice(h * 128, (h + 1) * 128)
        y_ref[j] = jnp.maximum(hl[:, sl] + bond_ref[a:a + 1, sl], 0.0)


def _ybuild_body(h_ref, bond_ref, y_ref):
    _write_y(y_ref, h_ref[...], bond_ref)


def _ybuild(h0, bond, n, emb):
    grid = (n // NB,)
    return pl.pallas_call(
        _ybuild_body,
        grid=grid,
        in_specs=[
            pl.BlockSpec((NB, emb), lambda i: (i, 0)),
            pl.BlockSpec((5, emb), lambda i: (0, 0)),
        ],
        out_specs=pl.BlockSpec((10, NB, 128), lambda i: (0, i, 0)),
        out_shape=jax.ShapeDtypeStruct((10, n, 128), F32),
    )(h0, bond)


def _t1b_body(z2_ref, ss_ref, hprev_ref, b_ref, vne_ref, bond_ref,
              hl_ref, y_ref):
    hnew = jnp.maximum(z2_ref[...] * ss_ref[0:1, :] + ss_ref[1:2, :], 0.0)
    hnew = hnew + hprev_ref[...]
    hl = hnew + jnp.dot(b_ref[...], vne_ref[...], preferred_element_type=F32)
    hl_ref[...] = hl
    _write_y(y_ref, hl, bond_ref)


def _t1b(z2, ss2, hprev, b, vne, bond, n, emb):
    grid = (n // NB,)
    return pl.pallas_call(
        _t1b_body,
        grid=grid,
        in_specs=[
            pl.BlockSpec((NB, emb), lambda i: (i, 0)),
            pl.BlockSpec((2, emb), lambda i: (0, 0)),
            pl.BlockSpec((NB, emb), lambda i: (i, 0)),
            pl.BlockSpec((NB, 128), lambda i: (i, 0)),
            pl.BlockSpec((128, emb), lambda i: (0, 0)),
            pl.BlockSpec((5, emb), lambda i: (0, 0)),
        ],
        out_specs=[
            pl.BlockSpec((NB, emb), lambda i: (i, 0)),
            pl.BlockSpec((10, NB, 128), lambda i: (0, i, 0)),
        ],
        out_shape=[
            jax.ShapeDtypeStruct((n, emb), F32),
            jax.ShapeDtypeStruct((10, n, 128), F32),
        ],
    )(z2, ss2, hprev, b, vne, bond)


def _mlp_stats_tail(i, nblk, n_rows, z, g_ref, bz_ref, ss_ref, acc_s, acc_q):
    s = jnp.sum(z, axis=0, keepdims=True)
    q = jnp.sum(z * z, axis=0, keepdims=True)

    @pl.when(i == 0)
    def _():
        acc_s[...] = s
        acc_q[...] = q

    @pl.when(i > 0)
    def _():
        acc_s[...] += s
        acc_q[...] += q

    @pl.when(i == nblk - 1)
    def _():
        mean = acc_s[...] * (1.0 / n_rows)
        var = acc_q[...] * (1.0 / n_rows) - mean * mean
        scale = lax.rsqrt(var + 1e-5) * g_ref[...]
        ss_ref[0:1, :] = scale
        ss_ref[1:2, :] = bz_ref[...] - mean * scale


def _p1_body(n_rows, hl_ref, agg_ref, eps_ref, w1_ref, g_ref, bz_ref,
             z1_ref, ss_ref, acc_s, acc_q):
    i = pl.program_id(0)
    nblk = pl.num_programs(0)
    aggr = jnp.concatenate([agg_ref[0], agg_ref[1]], axis=1)
    z = hl_ref[...] * eps_ref[...] + aggr
    z1 = jnp.dot(z, w1_ref[...], preferred_element_type=F32)
    z1_ref[...] = z1
    _mlp_stats_tail(i, nblk, n_rows, z1, g_ref, bz_ref, ss_ref, acc_s, acc_q)


def _p1(hl, aggr3, eps_row, w1, g1, bz1, n, emb):
    grid = (n // NB,)
    h = 2 * emb
    return pl.pallas_call(
        functools.partial(_p1_body, float(n)),
        grid=grid,
        in_specs=[
            pl.BlockSpec((NB, emb), lambda i: (i, 0)),
            pl.BlockSpec((2, NB, 128), lambda i: (0, i, 0)),
            pl.BlockSpec((1, emb), lambda i: (0, 0)),
            pl.BlockSpec((emb, h), lambda i: (0, 0)),
            pl.BlockSpec((1, h), lambda i: (0, 0)),
            pl.BlockSpec((1, h), lambda i: (0, 0)),
        ],
        out_specs=[
            pl.BlockSpec((NB, h), lambda i: (i, 0)),
            pl.BlockSpec((2, h), lambda i: (0, 0)),
        ],
        out_shape=[
            jax.ShapeDtypeStruct((n, h), F32),
            jax.ShapeDtypeStruct((2, h), F32),
        ],
        scratch_shapes=[pltpu.VMEM((1, h), F32), pltpu.VMEM((1, h), F32)],
    )(hl, aggr3, eps_row, w1, g1, bz1)


def _p2_body(n_rows, z1_ref, ss1_ref, w2_ref, g_ref, bz_ref,
             z2_ref, ss2_ref, acc_s, acc_q):
    i = pl.program_id(0)
    nblk = pl.num_programs(0)
    a = jnp.maximum(z1_ref[...] * ss1_ref[0:1, :] + ss1_ref[1:2, :], 0.0)
    z2 = jnp.dot(a, w2_ref[...], preferred_element_type=F32)
    z2_ref[...] = z2
    _mlp_stats_tail(i, nblk, n_rows, z2, g_ref, bz_ref, ss2_ref, acc_s, acc_q)


def _p2(z1, ss1, w2, g2, bz2, n, emb):
    grid = (n // NB,)
    h = 2 * emb
    return pl.pallas_call(
        functools.partial(_p2_body, float(n)),
        grid=grid,
        in_specs=[
            pl.BlockSpec((NB, h), lambda i: (i, 0)),
            pl.BlockSpec((2, h), lambda i: (0, 0)),
            pl.BlockSpec((h, emb), lambda i: (0, 0)),
            pl.BlockSpec((1, emb), lambda i: (0, 0)),
            pl.BlockSpec((1, emb), lambda i: (0, 0)),
        ],
        out_specs=[
            pl.BlockSpec((NB, emb), lambda i: (i, 0)),
            pl.BlockSpec((2, emb), lambda i: (0, 0)),
        ],
        out_shape=[
            jax.ShapeDtypeStruct((n, emb), F32),
            jax.ShapeDtypeStruct((2, emb), F32),
        ],
        scratch_shapes=[pltpu.VMEM((1, emb), F32), pltpu.VMEM((1, emb), F32)],
    )(z1, ss1, w2, g2, bz2)


def _bn_rows(t, g, b):
    mean = jnp.mean(t, axis=0, keepdims=True)
    var = jnp.mean(jnp.square(t - mean), axis=0, keepdims=True)
    return (t - mean) * lax.rsqrt(var + 1e-5) * g + b


def _vn_body(hl_ref, b_ref, vne_ref, wa_ref, ga_ref, bza_ref,
             wb_ref, gb_ref, bzb_ref, vout_ref, acc):
    i = pl.program_id(0)
    nblk = pl.num_programs(0)
    part = lax.dot_general(b_ref[...], hl_ref[...],
                           (((0,), (0,)), ((), ())),
                           preferred_element_type=F32)

    @pl.when(i == 0)
    def _():
        acc[...] = part

    @pl.when(i > 0)
    def _():
        acc[...] += part

    @pl.when(i == nblk - 1)
    def _():
        tmp = acc[...] + vne_ref[...]
        t = jnp.dot(tmp, wa_ref[...], preferred_element_type=F32)
        t = jnp.maximum(_bn_rows(t, ga_ref[...], bza_ref[...]), 0.0)
        t = jnp.dot(t, wb_ref[...], preferred_element_type=F32)
        t = jnp.maximum(_bn_rows(t, gb_ref[...], bzb_ref[...]), 0.0)
        vout_ref[...] = vne_ref[...] + t


def _vn(hl, b, vne, wa, ga, bza, wb, gb, bzb, n, emb):
    grid = (n // NB,)
    h = 2 * emb
    return pl.pallas_call(
        _vn_body,
        grid=grid,
        in_specs=[
            pl.BlockSpec((NB, emb), lambda i: (i, 0)),
            pl.BlockSpec((NB, 128), lambda i: (i, 0)),
            pl.BlockSpec((128, emb), lambda i: (0, 0)),
            pl.BlockSpec((emb, h), lambda i: (0, 0)),
            pl.BlockSpec((1, h), lambda i: (0, 0)),
            pl.BlockSpec((1, h), lambda i: (0, 0)),
            pl.BlockSpec((h, emb), lambda i: (0, 0)),
            pl.BlockSpec((1, emb), lambda i: (0, 0)),
            pl.BlockSpec((1, emb), lambda i: (0, 0)),
        ],
        out_specs=pl.BlockSpec((128, emb), lambda i: (0, 0)),
        out_shape=jax.ShapeDtypeStruct((128, emb), F32),
        scratch_shapes=[pltpu.VMEM((128, emb), F32)],
    )(hl, b, vne, wa, ga, bza, wb, gb, bzb)


def _final_body(z2_ref, ss2_ref, hl_ref, b_ref, pw_ref, pb_ref,
                out_ref, acc, cnt):
    i = pl.program_id(0)
    nblk = pl.num_programs(0)
    hnode = z2_ref[...] * ss2_ref[0:1, :] + ss2_ref[1:2, :] + hl_ref[...]
    bblk = b_ref[...]
    part = lax.dot_general(bblk, hnode, (((0,), (0,)), ((), ())),
                           preferred_element_type=F32)
    ones = jnp.ones(bblk.shape[:1] + (128,), F32)
    c = lax.dot_general(bblk, ones, (((0,), (0,)), ((), ())),
                        preferred_element_type=F32)

    @pl.when(i == 0)
    def _():
        acc[...] = part
        cnt[...] = c

    @pl.when(i > 0)
    def _():
        acc[...] += part
        cnt[...] += c

    @pl.when(i == nblk - 1)
    def _():
        hg = acc[...] / jnp.maximum(cnt[:, 0:1], 1.0)
        out_ref[...] = (jnp.dot(hg, pw_ref[...], preferred_element_type=F32)
                        + pb_ref[...])


def _final(z2, ss2, hl, b, pw, pb, n, emb, ncls):
    grid = (n // NB,)
    return pl.pallas_call(
        _final_body,
        grid=grid,
        in_specs=[
            pl.BlockSpec((NB, emb), lambda i: (i, 0)),
            pl.BlockSpec((2, emb), lambda i: (0, 0)),
            pl.BlockSpec((NB, emb), lambda i: (i, 0)),
            pl.BlockSpec((NB, 128), lambda i: (i, 0)),
            pl.BlockSpec((emb, ncls), lambda i: (0, 0)),
            pl.BlockSpec((1, ncls), lambda i: (0, 0)),
        ],
        out_specs=pl.BlockSpec((128, ncls), lambda i: (0, 0)),
        out_shape=jax.ShapeDtypeStruct((128, ncls), F32),
        scratch_shapes=[pltpu.VMEM((128, emb), F32), pltpu.VMEM((128, 128), F32)],
    )(z2, ss2, hl, b, pw, pb)


def _edge_aggr(y_flat, keys5, dst4, zeros, npad):
    no = keys5.shape[2]
    chunk = 128
    assert keys5.shape[3] == 8 and keys5.shape[4] == chunk
    rp = npad // 16
    assert rp % 8 == 0 and no % 2 == 0
    niter = no // 2
    mesh = plsc.VectorSubcoreMesh(core_axis_name="c", subcore_axis_name="s",
                                  num_cores=2, num_subcores=16)

    @functools.partial(
        pl.kernel, mesh=mesh,
        out_type=jax.ShapeDtypeStruct((2 * npad, 128), F32),
        scratch_types=[
            pltpu.VMEM((8, chunk), jnp.int32),
            pltpu.VMEM((8, chunk), jnp.int32),
            pltpu.VMEM((8, chunk), jnp.int32),
            pltpu.VMEM((8, chunk), jnp.int32),
            pltpu.VMEM((chunk, 128), F32),
            pltpu.VMEM((chunk, 128), F32),
            pltpu.VMEM_SHARED((npad, 128), F32),
            pltpu.SemaphoreType.DMA,
            pltpu.SemaphoreType.DMA,
            pltpu.SemaphoreType.DMA,
        ],
    )
    def k(y_hbm, keys_hbm, dst_hbm, zeros_hbm, out_hbm,
          koa, kob, doa, dob, ra, rb, acc, sa, sb, sz):
        ko = [koa, kob]
        do = [doa, dob]
        rows = [ra, rb]
        sems = [sa, sb]
        c = lax.axis_index("c")
        s = lax.axis_index("s")

        pltpu.async_copy(zeros_hbm.at[pl.ds(s * rp, rp)],
                         acc.at[pl.ds(s * rp, rp)], sz)

        def load_oct(q, b):
            pltpu.sync_copy(keys_hbm.at[c, s, q], ko[b])
            pltpu.sync_copy(dst_hbm.at[s, q], do[b])

        load_oct(0, 0)
        pltpu.make_async_copy(zeros_hbm.at[pl.ds(s * rp, rp)],
                              acc.at[pl.ds(s * rp, rp)], sz).wait()
        plsc.subcore_barrier()
        pltpu.async_copy(y_hbm.at[ko[0].at[0]], rows[0], sems[0])

        def body(i, carry):
            load_oct(2 * i + 1, 1)
            for j in range(16):
                ob = j // 8
                row = j % 8
                rpar = j % 2
                if j < 15:
                    pltpu.async_copy(y_hbm.at[ko[(j + 1) // 8].at[(j + 1) % 8]],
                                     rows[1 - rpar], sems[1 - rpar])
                if j == 13:
                    @pl.when(i + 1 < niter)
                    def _():
                        load_oct(2 * i + 2, 0)
                if j == 15:
                    @pl.when(i + 1 < niter)
                    def _():
                        pltpu.async_copy(y_hbm.at[ko[0].at[0]],
                                         rows[0], sems[0])
                pltpu.make_async_copy(y_hbm.at[ko[ob].at[row]],
                                      rows[rpar], sems[rpar]).wait()
                pltpu.sync_copy(rows[rpar], acc.at[do[ob].at[row]], add=True)
            return carry

        lax.fori_loop(0, niter, body, 0)
        plsc.subcore_barrier()
        pltpu.sync_copy(acc.at[pl.ds(s * rp, rp)],
                        out_hbm.at[pl.ds(c * npad + s * rp, rp)])

    return k(y_flat, keys5, dst4, zeros)


def kernel(x, edge_index, edge_attr, batch, params):
    n = x.shape[0]
    e = edge_attr.shape[0]
    emb = params["atom_table"].shape[1]
    ncls = params["pred_W"].shape[1]
    nlayer = len(params["convs"])
    assert n % NB == 0 and emb == 256 and e % 16 == 0

    npad = ((n // 16 + 127) // 128) * 128 * 16

    x2 = x.astype(jnp.int32).reshape(n, 1)
    batch2 = batch.astype(jnp.int32).reshape(n, 1)
    vocab = params["atom_table"].shape[0]
    table_pad = jnp.concatenate(
        [params["atom_table"], jnp.zeros((128 - vocab, emb), F32)], axis=0)

    src = edge_index[0].astype(jnp.int32)
    dst = edge_index[1].astype(jnp.int32)
    ep = e // 16
    oct_e = 1024
    no = -(-ep // (2 * oct_e)) * 2
    epp = no * oct_e
    keys = edge_attr.astype(jnp.int32) * n + src
    keys_t = jnp.pad(keys.reshape(16, ep), ((0, 0), (0, epp - ep)))
    keys5 = jnp.stack([keys_t, keys_t + 5 * n]).reshape(2, 16, no, 8, 128)
    pad_dst = n + (jnp.arange(epp - ep, dtype=jnp.int32) % (npad - n))
    dst4 = jnp.concatenate(
        [dst.reshape(16, ep), jnp.broadcast_to(pad_dst, (16, epp - ep))],
        axis=1).reshape(16, no, 8, 128)
    zeros = jnp.zeros((npad, 128), F32)

    b_onehot, h0 = _prep(x2, batch2, table_pad, n)

    hl = h0
    z2 = ss2 = None
    vne = None
    for l in range(nlayer):
        p = params["convs"][l]
        bond = p["bond_table"]
        if l == 0:
            y = _ybuild(hl, bond, n, emb)
        else:
            hl, y = _t1b(z2, ss2, hl, b_onehot, vne, bond, n, emb)
        aggr = _edge_aggr(y.reshape(10 * n, 128), keys5, dst4, zeros, npad)
        eps_row = (1.0 + p["eps"]) * jnp.ones((1, emb), F32)
        z1, ss1 = _p1(hl, aggr.reshape(2, npad, 128), eps_row, p["W1"],
                      p["bn1_g"].reshape(1, -1), p["bn1_b"].reshape(1, -1),
                      n, emb)
        z2, ss2 = _p2(z1, ss1, p["W2"],
                      p["bn_g"].reshape(1, -1), p["bn_b"].reshape(1, -1),
                      n, emb)
        if l < nlayer - 1:
            mp = params["vn_mlps"][l]
            vne_in = vne if vne is not None else jnp.zeros((128, emb), F32)
            vne = _vn(hl, b_onehot, vne_in, mp["Wa"],
                      mp["bn_a_g"].reshape(1, -1), mp["bn_a_b"].reshape(1, -1),
                      mp["Wb"],
                      mp["bn_b_g"].reshape(1, -1), mp["bn_b_b"].reshape(1, -1),
                      n, emb)

    return _final(z2, ss2, hl, b_onehot, params["pred_W"],
                  params["pred_b"].reshape(1, -1), n, emb, ncls)


# speedup vs baseline: 1.0180x; 1.0027x over previous
import functools

import jax
import jax.numpy as jnp
from jax import lax
from jax.experimental import pallas as pl
from jax.experimental.pallas import tpu as pltpu
from jax.experimental.pallas import tpu_sc as plsc

F32 = jnp.float32
NB = 1000


def _prep_body(x_ref, batch_ref, table_ref, b_ref, h0_ref):
    nb = x_ref.shape[0]
    iota_g = lax.broadcasted_iota(jnp.int32, (nb, 128), 1)
    b_ref[...] = (iota_g == batch_ref[...]).astype(F32)
    onehot_x = (iota_g == x_ref[...]).astype(F32)
    h0_ref[...] = jnp.dot(onehot_x, table_ref[...], preferred_element_type=F32)


def _prep(x2, batch2, table_pad, n):
    grid = (n // NB,)
    return pl.pallas_call(
        _prep_body,
        grid=grid,
        in_specs=[
            pl.BlockSpec((NB, 1), lambda i: (i, 0)),
            pl.BlockSpec((NB, 1), lambda i: (i, 0)),
            pl.BlockSpec(table_pad.shape, lambda i: (0, 0)),
        ],
        out_specs=[
            pl.BlockSpec((NB, 128), lambda i: (i, 0)),
            pl.BlockSpec((NB, table_pad.shape[1]), lambda i: (i, 0)),
        ],
        out_shape=[
            jax.ShapeDtypeStruct((n, 128), F32),
            jax.ShapeDtypeStruct((n, table_pad.shape[1]), F32),
        ],
    )(x2, batch2, table_pad)


def _write_y(y_ref, hl, bond_ref):
    for j in range(10):
        h, a = divmod(j, 5)
        sl = slice(h * 128, (h + 1) * 128)
        y_ref[j] = jnp.maximum(hl[:, sl] + bond_ref[a:a + 1, sl], 0.0)


def _ybuild_body(h_ref, bond_ref, y_ref):
    _write_y(y_ref, h_ref[...], bond_ref)


def _ybuild(h0, bond, n, emb):
    grid = (n // NB,)
    return pl.pallas_call(
        _ybuild_body,
        grid=grid,
        in_specs=[
            pl.BlockSpec((NB, emb), lambda i: (i, 0)),
            pl.BlockSpec((5, emb), lambda i: (0, 0)),
        ],
        out_specs=pl.BlockSpec((10, NB, 128), lambda i: (0, i, 0)),
        out_shape=jax.ShapeDtypeStruct((10, n, 128), F32),
    )(h0, bond)


def _t1b_body(z2_ref, ss_ref, hprev_ref, b_ref, vne_ref, bond_ref,
              hl_ref, y_ref):
    hnew = jnp.maximum(z2_ref[...] * ss_ref[0:1, :] + ss_ref[1:2, :], 0.0)
    hnew = hnew + hprev_ref[...]
    hl = hnew + jnp.dot(b_ref[...], vne_ref[...], preferred_element_type=F32)
    hl_ref[...] = hl
    _write_y(y_ref, hl, bond_ref)


def _t1b(z2, ss2, hprev, b, vne, bond, n, emb):
    grid = (n // NB,)
    return pl.pallas_call(
        _t1b_body,
        grid=grid,
        in_specs=[
            pl.BlockSpec((NB, emb), lambda i: (i, 0)),
            pl.BlockSpec((2, emb), lambda i: (0, 0)),
            pl.BlockSpec((NB, emb), lambda i: (i, 0)),
            pl.BlockSpec((NB, 128), lambda i: (i, 0)),
            pl.BlockSpec((128, emb), lambda i: (0, 0)),
            pl.BlockSpec((5, emb), lambda i: (0, 0)),
        ],
        out_specs=[
            pl.BlockSpec((NB, emb), lambda i: (i, 0)),
            pl.BlockSpec((10, NB, 128), lambda i: (0, i, 0)),
        ],
        out_shape=[
            jax.ShapeDtypeStruct((n, emb), F32),
            jax.ShapeDtypeStruct((10, n, 128), F32),
        ],
    )(z2, ss2, hprev, b, vne, bond)


def _mlp_stats_tail(i, nblk, n_rows, z, g_ref, bz_ref, ss_ref, acc_s, acc_q):
    s = jnp.sum(z, axis=0, keepdims=True)
    q = jnp.sum(z * z, axis=0, keepdims=True)

    @pl.when(i == 0)
    def _():
        acc_s[...] = s
        acc_q[...] = q

    @pl.when(i > 0)
    def _():
        acc_s[...] += s
        acc_q[...] += q

    @pl.when(i == nblk - 1)
    def _():
        mean = acc_s[...] * (1.0 / n_rows)
        var = acc_q[...] * (1.0 / n_rows) - mean * mean
        scale = lax.rsqrt(var + 1e-5) * g_ref[...]
        ss_ref[0:1, :] = scale
        ss_ref[1:2, :] = bz_ref[...] - mean * scale


def _p1_body(n_rows, hl_ref, agg_ref, eps_ref, w1_ref, g_ref, bz_ref,
             z1_ref, ss_ref, acc_s, acc_q):
    i = pl.program_id(0)
    nblk = pl.num_programs(0)
    aggr = jnp.concatenate([agg_ref[0], agg_ref[1]], axis=1)
    z = hl_ref[...] * eps_ref[...] + aggr
    z1 = jnp.dot(z, w1_ref[...], preferred_element_type=F32)
    z1_ref[...] = z1
    _mlp_stats_tail(i, nblk, n_rows, z1, g_ref, bz_ref, ss_ref, acc_s, acc_q)


def _p1(hl, aggr3, eps_row, w1, g1, bz1, n, emb):
    grid = (n // NB,)
    h = 2 * emb
    return pl.pallas_call(
        functools.partial(_p1_body, float(n)),
        grid=grid,
        in_specs=[
            pl.BlockSpec((NB, emb), lambda i: (i, 0)),
            pl.BlockSpec((2, NB, 128), lambda i: (0, i, 0)),
            pl.BlockSpec((1, emb), lambda i: (0, 0)),
            pl.BlockSpec((emb, h), lambda i: (0, 0)),
            pl.BlockSpec((1, h), lambda i: (0, 0)),
            pl.BlockSpec((1, h), lambda i: (0, 0)),
        ],
        out_specs=[
            pl.BlockSpec((NB, h), lambda i: (i, 0)),
            pl.BlockSpec((2, h), lambda i: (0, 0)),
        ],
        out_shape=[
            jax.ShapeDtypeStruct((n, h), F32),
            jax.ShapeDtypeStruct((2, h), F32),
        ],
        scratch_shapes=[pltpu.VMEM((1, h), F32), pltpu.VMEM((1, h), F32)],
    )(hl, aggr3, eps_row, w1, g1, bz1)


def _p2_body(n_rows, z1_ref, ss1_ref, w2_ref, g_ref, bz_ref,
             z2_ref, ss2_ref, acc_s, acc_q):
    i = pl.program_id(0)
    nblk = pl.num_programs(0)
    a = jnp.maximum(z1_ref[...] * ss1_ref[0:1, :] + ss1_ref[1:2, :], 0.0)
    z2 = jnp.dot(a, w2_ref[...], preferred_element_type=F32)
    z2_ref[...] = z2
    _mlp_stats_tail(i, nblk, n_rows, z2, g_ref, bz_ref, ss2_ref, acc_s, acc_q)


def _p2(z1, ss1, w2, g2, bz2, n, emb):
    grid = (n // NB,)
    h = 2 * emb
    return pl.pallas_call(
        functools.partial(_p2_body, float(n)),
        grid=grid,
        in_specs=[
            pl.BlockSpec((NB, h), lambda i: (i, 0)),
            pl.BlockSpec((2, h), lambda i: (0, 0)),
            pl.BlockSpec((h, emb), lambda i: (0, 0)),
            pl.BlockSpec((1, emb), lambda i: (0, 0)),
            pl.BlockSpec((1, emb), lambda i: (0, 0)),
        ],
        out_specs=[
            pl.BlockSpec((NB, emb), lambda i: (i, 0)),
            pl.BlockSpec((2, emb), lambda i: (0, 0)),
        ],
        out_shape=[
            jax.ShapeDtypeStruct((n, emb), F32),
            jax.ShapeDtypeStruct((2, emb), F32),
        ],
        scratch_shapes=[pltpu.VMEM((1, emb), F32), pltpu.VMEM((1, emb), F32)],
    )(z1, ss1, w2, g2, bz2)


def _bn_rows(t, g, b):
    mean = jnp.mean(t, axis=0, keepdims=True)
    var = jnp.mean(jnp.square(t - mean), axis=0, keepdims=True)
    return (t - mean) * lax.rsqrt(var + 1e-5) * g + b


def _vn_body(hl_ref, b_ref, vne_ref, wa_ref, ga_ref, bza_ref,
             wb_ref, gb_ref, bzb_ref, vout_ref, acc):
    i = pl.program_id(0)
    nblk = pl.num_programs(0)
    part = lax.dot_general(b_ref[...], hl_ref[...],
                           (((0,), (0,)), ((), ())),
                           preferred_element_type=F32)

    @pl.when(i == 0)
    def _():
        acc[...] = part

    @pl.when(i > 0)
    def _():
        acc[...] += part

    @pl.when(i == nblk - 1)
    def _():
        tmp = acc[...] + vne_ref[...]
        t = jnp.dot(tmp, wa_ref[...], preferred_element_type=F32)
        t = jnp.maximum(_bn_rows(t, ga_ref[...], bza_ref[...]), 0.0)
        t = jnp.dot(t, wb_ref[...], preferred_element_type=F32)
        t = jnp.maximum(_bn_rows(t, gb_ref[...], bzb_ref[...]), 0.0)
        vout_ref[...] = vne_ref[...] + t


def _vn(hl, b, vne, wa, ga, bza, wb, gb, bzb, n, emb):
    grid = (n // NB,)
    h = 2 * emb
    return pl.pallas_call(
        _vn_body,
        grid=grid,
        in_specs=[
            pl.BlockSpec((NB, emb), lambda i: (i, 0)),
            pl.BlockSpec((NB, 128), lambda i: (i, 0)),
            pl.BlockSpec((128, emb), lambda i: (0, 0)),
            pl.BlockSpec((emb, h), lambda i: (0, 0)),
            pl.BlockSpec((1, h), lambda i: (0, 0)),
            pl.BlockSpec((1, h), lambda i: (0, 0)),
            pl.BlockSpec((h, emb), lambda i: (0, 0)),
            pl.BlockSpec((1, emb), lambda i: (0, 0)),
            pl.BlockSpec((1, emb), lambda i: (0, 0)),
        ],
        out_specs=pl.BlockSpec((128, emb), lambda i: (0, 0)),
        out_shape=jax.ShapeDtypeStruct((128, emb), F32),
        scratch_shapes=[pltpu.VMEM((128, emb), F32)],
    )(hl, b, vne, wa, ga, bza, wb, gb, bzb)


def _final_body(z2_ref, ss2_ref, hl_ref, b_ref, pw_ref, pb_ref,
                out_ref, acc, cnt):
    i = pl.program_id(0)
    nblk = pl.num_programs(0)
    hnode = z2_ref[...] * ss2_ref[0:1, :] + ss2_ref[1:2, :] + hl_ref[...]
    bblk = b_ref[...]
    part = lax.dot_general(bblk, hnode, (((0,), (0,)), ((), ())),
                           preferred_element_type=F32)
    ones = jnp.ones(bblk.shape[:1] + (128,), F32)
    c = lax.dot_general(bblk, ones, (((0,), (0,)), ((), ())),
                        preferred_element_type=F32)

    @pl.when(i == 0)
    def _():
        acc[...] = part
        cnt[...] = c

    @pl.when(i > 0)
    def _():
        acc[...] += part
        cnt[...] += c

    @pl.when(i == nblk - 1)
    def _():
        hg = acc[...] / jnp.maximum(cnt[:, 0:1], 1.0)
        out_ref[...] = (jnp.dot(hg, pw_ref[...], preferred_element_type=F32)
                        + pb_ref[...])


def _final(z2, ss2, hl, b, pw, pb, n, emb, ncls):
    grid = (n // NB,)
    return pl.pallas_call(
        _final_body,
        grid=grid,
        in_specs=[
            pl.BlockSpec((NB, emb), lambda i: (i, 0)),
            pl.BlockSpec((2, emb), lambda i: (0, 0)),
            pl.BlockSpec((NB, emb), lambda i: (i, 0)),
            pl.BlockSpec((NB, 128), lambda i: (i, 0)),
            pl.BlockSpec((emb, ncls), lambda i: (0, 0)),
            pl.BlockSpec((1, ncls), lambda i: (0, 0)),
        ],
        out_specs=pl.BlockSpec((128, ncls), lambda i: (0, 0)),
        out_shape=jax.ShapeDtypeStruct((128, ncls), F32),
        scratch_shapes=[pltpu.VMEM((128, emb), F32), pltpu.VMEM((128, 128), F32)],
    )(z2, ss2, hl, b, pw, pb)


def _edge_aggr(y_flat, keys5, dst4, zeros, npad):
    no = keys5.shape[2]
    chunk = 128
    assert keys5.shape[3] == 8 and keys5.shape[4] == chunk
    rp = npad // 16
    assert rp % 8 == 0 and no % 2 == 0
    niter = no // 2
    mesh = plsc.VectorSubcoreMesh(core_axis_name="c", subcore_axis_name="s",
                                  num_cores=2, num_subcores=16)

    @functools.partial(
        pl.kernel, mesh=mesh,
        out_type=jax.ShapeDtypeStruct((2 * npad, 128), F32),
        scratch_types=[
            pltpu.VMEM((8, chunk), jnp.int32),
            pltpu.VMEM((8, chunk), jnp.int32),
            pltpu.VMEM((8, chunk), jnp.int32),
            pltpu.VMEM((8, chunk), jnp.int32),
            pltpu.VMEM((chunk, 128), F32),
            pltpu.VMEM((chunk, 128), F32),
            pltpu.VMEM((32, 128), F32),
            pltpu.VMEM_SHARED((npad, 128), F32),
            pltpu.SemaphoreType.DMA,
            pltpu.SemaphoreType.DMA,
            pltpu.SemaphoreType.DMA,
        ],
    )
    def k(y_hbm, keys_hbm, dst_hbm, zeros_hbm, out_hbm,
          koa, kob, doa, dob, ra, rb, zbuf, acc, sa, sb, sz):
        ko = [koa, kob]
        do = [doa, dob]
        rows = [ra, rb]
        sems = [sa, sb]
        c = lax.axis_index("c")
        s = lax.axis_index("s")

        def zrow(i, carry):
            for t in range(8):
                zbuf[i, pl.ds(t * 16, 16)] = jnp.zeros((16,), F32)
            return carry

        lax.fori_loop(0, 32, zrow, 0)
        for t in range(rp // 32):
            pltpu.sync_copy(zbuf, acc.at[pl.ds(s * rp + t * 32, 32)])

        def load_oct(q, b):
            pltpu.sync_copy(keys_hbm.at[c, s, q], ko[b])
            pltpu.sync_copy(dst_hbm.at[s, q], do[b])

        load_oct(0, 0)
        plsc.subcore_barrier()
        pltpu.async_copy(y_hbm.at[ko[0].at[0]], rows[0], sems[0])

        def body(i, carry):
            load_oct(2 * i + 1, 1)
            for j in range(16):
                ob = j // 8
                row = j % 8
                rpar = j % 2
                if j < 15:
                    pltpu.async_copy(y_hbm.at[ko[(j + 1) // 8].at[(j + 1) % 8]],
                                     rows[1 - rpar], sems[1 - rpar])
                if j == 13:
                    @pl.when(i + 1 < niter)
                    def _():
                        load_oct(2 * i + 2, 0)
                if j == 15:
                    @pl.when(i + 1 < niter)
                    def _():
                        pltpu.async_copy(y_hbm.at[ko[0].at[0]],
                                         rows[0], sems[0])
                pltpu.make_async_copy(y_hbm.at[ko[ob].at[row]],
                                      rows[rpar], sems[rpar]).wait()
                pltpu.sync_copy(rows[rpar], acc.at[do[ob].at[row]], add=True)
            return carry

        lax.fori_loop(0, niter, body, 0)
        plsc.subcore_barrier()
        pltpu.sync_copy(acc.at[pl.ds(s * rp, rp)],
                        out_hbm.at[pl.ds(c * npad + s * rp, rp)])

    return k(y_flat, keys5, dst4, zeros)


def kernel(x, edge_index, edge_attr, batch, params):
    n = x.shape[0]
    e = edge_attr.shape[0]
    emb = params["atom_table"].shape[1]
    ncls = params["pred_W"].shape[1]
    nlayer = len(params["convs"])
    assert n % NB == 0 and emb == 256 and e % 16 == 0

    npad = ((n // 16 + 127) // 128) * 128 * 16

    x2 = x.astype(jnp.int32).reshape(n, 1)
    batch2 = batch.astype(jnp.int32).reshape(n, 1)
    vocab = params["atom_table"].shape[0]
    table_pad = jnp.concatenate(
        [params["atom_table"], jnp.zeros((128 - vocab, emb), F32)], axis=0)

    src = edge_index[0].astype(jnp.int32)
    dst = edge_index[1].astype(jnp.int32)
    ep = e // 16
    oct_e = 1024
    no = -(-ep // (2 * oct_e)) * 2
    epp = no * oct_e
    keys = edge_attr.astype(jnp.int32) * n + src
    keys_t = jnp.pad(keys.reshape(16, ep), ((0, 0), (0, epp - ep)))
    keys5 = jnp.stack([keys_t, keys_t + 5 * n]).reshape(2, 16, no, 8, 128)
    pad_dst = n + (jnp.arange(epp - ep, dtype=jnp.int32) % (npad - n))
    dst4 = jnp.concatenate(
        [dst.reshape(16, ep), jnp.broadcast_to(pad_dst, (16, epp - ep))],
        axis=1).reshape(16, no, 8, 128)
    zeros = jnp.zeros((npad, 128), F32)

    b_onehot, h0 = _prep(x2, batch2, table_pad, n)

    hl = h0
    z2 = ss2 = None
    vne = None
    for l in range(nlayer):
        p = params["convs"][l]
        bond = p["bond_table"]
        if l == 0:
            y = _ybuild(hl, bond, n, emb)
        else:
            hl, y = _t1b(z2, ss2, hl, b_onehot, vne, bond, n, emb)
        aggr = _edge_aggr(y.reshape(10 * n, 128), keys5, dst4, zeros, npad)
        eps_row = (1.0 + p["eps"]) * jnp.ones((1, emb), F32)
        z1, ss1 = _p1(hl, aggr.reshape(2, npad, 128), eps_row, p["W1"],
                      p["bn1_g"].reshape(1, -1), p["bn1_b"].reshape(1, -1),
                      n, emb)
        z2, ss2 = _p2(z1, ss1, p["W2"],
                      p["bn_g"].reshape(1, -1), p["bn_b"].reshape(1, -1),
                      n, emb)
        if l < nlayer - 1:
            mp = params["vn_mlps"][l]
            vne_in = vne if vne is not None else jnp.zeros((128, emb), F32)
            vne = _vn(hl, b_onehot, vne_in, mp["Wa"],
                      mp["bn_a_g"].reshape(1, -1), mp["bn_a_b"].reshape(1, -1),
                      mp["Wb"],
                      mp["bn_b_g"].reshape(1, -1), mp["bn_b_b"].reshape(1, -1),
                      n, emb)

    return _final(z2, ss2, hl, b_onehot, params["pred_W"],
                  params["pred_b"].reshape(1, -1), n, emb, ncls)


# speedup vs baseline: 1.5116x; 1.4848x over previous
import functools

import jax
import jax.numpy as jnp
from jax import lax
from jax.experimental import pallas as pl
from jax.experimental.pallas import tpu as pltpu
from jax.experimental.pallas import tpu_sc as plsc

F32 = jnp.float32
NB = 1000


def _prep_body(x_ref, batch_ref, table_ref, b_ref, h0_ref):
    nb = x_ref.shape[0]
    iota_g = lax.broadcasted_iota(jnp.int32, (nb, 128), 1)
    b_ref[...] = (iota_g == batch_ref[...]).astype(F32)
    onehot_x = (iota_g == x_ref[...]).astype(F32)
    h0_ref[...] = jnp.dot(onehot_x, table_ref[...], preferred_element_type=F32)


def _prep(x2, batch2, table_pad, n):
    grid = (n // NB,)
    return pl.pallas_call(
        _prep_body,
        grid=grid,
        in_specs=[
            pl.BlockSpec((NB, 1), lambda i: (i, 0)),
            pl.BlockSpec((NB, 1), lambda i: (i, 0)),
            pl.BlockSpec(table_pad.shape, lambda i: (0, 0)),
        ],
        out_specs=[
            pl.BlockSpec((NB, 128), lambda i: (i, 0)),
            pl.BlockSpec((NB, table_pad.shape[1]), lambda i: (i, 0)),
        ],
        out_shape=[
            jax.ShapeDtypeStruct((n, 128), F32),
            jax.ShapeDtypeStruct((n, table_pad.shape[1]), F32),
        ],
    )(x2, batch2, table_pad)


def _write_y(y_ref, hl, bond_ref):
    for j in range(10):
        h, a = divmod(j, 5)
        sl = slice(h * 128, (h + 1) * 128)
        y_ref[j] = jnp.maximum(hl[:, sl] + bond_ref[a:a + 1, sl], 0.0)


def _ybuild_body(h_ref, bond_ref, y_ref):
    _write_y(y_ref, h_ref[...], bond_ref)


def _ybuild(h0, bond, n, emb):
    grid = (n // NB,)
    return pl.pallas_call(
        _ybuild_body,
        grid=grid,
        in_specs=[
            pl.BlockSpec((NB, emb), lambda i: (i, 0)),
            pl.BlockSpec((5, emb), lambda i: (0, 0)),
        ],
        out_specs=pl.BlockSpec((10, NB, 128), lambda i: (0, i, 0)),
        out_shape=jax.ShapeDtypeStruct((10, n, 128), F32),
    )(h0, bond)


def _t1b_body(z2_ref, ss_ref, hprev_ref, b_ref, vne_ref, bond_ref,
              hl_ref, y_ref):
    hnew = jnp.maximum(z2_ref[...] * ss_ref[0:1, :] + ss_ref[1:2, :], 0.0)
    hnew = hnew + hprev_ref[...]
    hl = hnew + jnp.dot(b_ref[...], vne_ref[...], preferred_element_type=F32)
    hl_ref[...] = hl
    _write_y(y_ref, hl, bond_ref)


def _t1b(z2, ss2, hprev, b, vne, bond, n, emb):
    grid = (n // NB,)
    return pl.pallas_call(
        _t1b_body,
        grid=grid,
        in_specs=[
            pl.BlockSpec((NB, emb), lambda i: (i, 0)),
            pl.BlockSpec((2, emb), lambda i: (0, 0)),
            pl.BlockSpec((NB, emb), lambda i: (i, 0)),
            pl.BlockSpec((NB, 128), lambda i: (i, 0)),
            pl.BlockSpec((128, emb), lambda i: (0, 0)),
            pl.BlockSpec((5, emb), lambda i: (0, 0)),
        ],
        out_specs=[
            pl.BlockSpec((NB, emb), lambda i: (i, 0)),
            pl.BlockSpec((10, NB, 128), lambda i: (0, i, 0)),
        ],
        out_shape=[
            jax.ShapeDtypeStruct((n, emb), F32),
            jax.ShapeDtypeStruct((10, n, 128), F32),
        ],
    )(z2, ss2, hprev, b, vne, bond)


def _mlp_stats_tail(i, nblk, n_rows, z, g_ref, bz_ref, ss_ref, acc_s, acc_q):
    s = jnp.sum(z, axis=0, keepdims=True)
    q = jnp.sum(z * z, axis=0, keepdims=True)

    @pl.when(i == 0)
    def _():
        acc_s[...] = s
        acc_q[...] = q

    @pl.when(i > 0)
    def _():
        acc_s[...] += s
        acc_q[...] += q

    @pl.when(i == nblk - 1)
    def _():
        mean = acc_s[...] * (1.0 / n_rows)
        var = acc_q[...] * (1.0 / n_rows) - mean * mean
        scale = lax.rsqrt(var + 1e-5) * g_ref[...]
        ss_ref[0:1, :] = scale
        ss_ref[1:2, :] = bz_ref[...] - mean * scale


def _p1_body(n_rows, hl_ref, agg_ref, eps_ref, w1_ref, g_ref, bz_ref,
             z1_ref, ss_ref, acc_s, acc_q):
    i = pl.program_id(0)
    nblk = pl.num_programs(0)
    aggr = jnp.concatenate([agg_ref[0], agg_ref[1]], axis=1)
    z = hl_ref[...] * eps_ref[...] + aggr
    z1 = jnp.dot(z, w1_ref[...], preferred_element_type=F32)
    z1_ref[...] = z1
    _mlp_stats_tail(i, nblk, n_rows, z1, g_ref, bz_ref, ss_ref, acc_s, acc_q)


def _p1(hl, aggr3, eps_row, w1, g1, bz1, n, emb):
    grid = (n // NB,)
    h = 2 * emb
    return pl.pallas_call(
        functools.partial(_p1_body, float(n)),
        grid=grid,
        in_specs=[
            pl.BlockSpec((NB, emb), lambda i: (i, 0)),
            pl.BlockSpec((2, NB, 128), lambda i: (0, i, 0)),
            pl.BlockSpec((1, emb), lambda i: (0, 0)),
            pl.BlockSpec((emb, h), lambda i: (0, 0)),
            pl.BlockSpec((1, h), lambda i: (0, 0)),
            pl.BlockSpec((1, h), lambda i: (0, 0)),
        ],
        out_specs=[
            pl.BlockSpec((NB, h), lambda i: (i, 0)),
            pl.BlockSpec((2, h), lambda i: (0, 0)),
        ],
        out_shape=[
            jax.ShapeDtypeStruct((n, h), F32),
            jax.ShapeDtypeStruct((2, h), F32),
        ],
        scratch_shapes=[pltpu.VMEM((1, h), F32), pltpu.VMEM((1, h), F32)],
    )(hl, aggr3, eps_row, w1, g1, bz1)


def _p2_body(n_rows, z1_ref, ss1_ref, w2_ref, g_ref, bz_ref,
             z2_ref, ss2_ref, acc_s, acc_q):
    i = pl.program_id(0)
    nblk = pl.num_programs(0)
    a = jnp.maximum(z1_ref[...] * ss1_ref[0:1, :] + ss1_ref[1:2, :], 0.0)
    z2 = jnp.dot(a, w2_ref[...], preferred_element_type=F32)
    z2_ref[...] = z2
    _mlp_stats_tail(i, nblk, n_rows, z2, g_ref, bz_ref, ss2_ref, acc_s, acc_q)


def _p2(z1, ss1, w2, g2, bz2, n, emb):
    grid = (n // NB,)
    h = 2 * emb
    return pl.pallas_call(
        functools.partial(_p2_body, float(n)),
        grid=grid,
        in_specs=[
            pl.BlockSpec((NB, h), lambda i: (i, 0)),
            pl.BlockSpec((2, h), lambda i: (0, 0)),
            pl.BlockSpec((h, emb), lambda i: (0, 0)),
            pl.BlockSpec((1, emb), lambda i: (0, 0)),
            pl.BlockSpec((1, emb), lambda i: (0, 0)),
        ],
        out_specs=[
            pl.BlockSpec((NB, emb), lambda i: (i, 0)),
            pl.BlockSpec((2, emb), lambda i: (0, 0)),
        ],
        out_shape=[
            jax.ShapeDtypeStruct((n, emb), F32),
            jax.ShapeDtypeStruct((2, emb), F32),
        ],
        scratch_shapes=[pltpu.VMEM((1, emb), F32), pltpu.VMEM((1, emb), F32)],
    )(z1, ss1, w2, g2, bz2)


def _bn_rows(t, g, b):
    mean = jnp.mean(t, axis=0, keepdims=True)
    var = jnp.mean(jnp.square(t - mean), axis=0, keepdims=True)
    return (t - mean) * lax.rsqrt(var + 1e-5) * g + b


def _vn_body(hl_ref, b_ref, vne_ref, wa_ref, ga_ref, bza_ref,
             wb_ref, gb_ref, bzb_ref, vout_ref, acc):
    i = pl.program_id(0)
    nblk = pl.num_programs(0)
    part = lax.dot_general(b_ref[...], hl_ref[...],
                           (((0,), (0,)), ((), ())),
                           preferred_element_type=F32)

    @pl.when(i == 0)
    def _():
        acc[...] = part

    @pl.when(i > 0)
    def _():
        acc[...] += part

    @pl.when(i == nblk - 1)
    def _():
        tmp = acc[...] + vne_ref[...]
        t = jnp.dot(tmp, wa_ref[...], preferred_element_type=F32)
        t = jnp.maximum(_bn_rows(t, ga_ref[...], bza_ref[...]), 0.0)
        t = jnp.dot(t, wb_ref[...], preferred_element_type=F32)
        t = jnp.maximum(_bn_rows(t, gb_ref[...], bzb_ref[...]), 0.0)
        vout_ref[...] = vne_ref[...] + t


def _vn(hl, b, vne, wa, ga, bza, wb, gb, bzb, n, emb):
    grid = (n // NB,)
    h = 2 * emb
    return pl.pallas_call(
        _vn_body,
        grid=grid,
        in_specs=[
            pl.BlockSpec((NB, emb), lambda i: (i, 0)),
            pl.BlockSpec((NB, 128), lambda i: (i, 0)),
            pl.BlockSpec((128, emb), lambda i: (0, 0)),
            pl.BlockSpec((emb, h), lambda i: (0, 0)),
            pl.BlockSpec((1, h), lambda i: (0, 0)),
            pl.BlockSpec((1, h), lambda i: (0, 0)),
            pl.BlockSpec((h, emb), lambda i: (0, 0)),
            pl.BlockSpec((1, emb), lambda i: (0, 0)),
            pl.BlockSpec((1, emb), lambda i: (0, 0)),
        ],
        out_specs=pl.BlockSpec((128, emb), lambda i: (0, 0)),
        out_shape=jax.ShapeDtypeStruct((128, emb), F32),
        scratch_shapes=[pltpu.VMEM((128, emb), F32)],
    )(hl, b, vne, wa, ga, bza, wb, gb, bzb)


def _final_body(z2_ref, ss2_ref, hl_ref, b_ref, pw_ref, pb_ref,
                out_ref, acc, cnt):
    i = pl.program_id(0)
    nblk = pl.num_programs(0)
    hnode = z2_ref[...] * ss2_ref[0:1, :] + ss2_ref[1:2, :] + hl_ref[...]
    bblk = b_ref[...]
    part = lax.dot_general(bblk, hnode, (((0,), (0,)), ((), ())),
                           preferred_element_type=F32)
    ones = jnp.ones(bblk.shape[:1] + (128,), F32)
    c = lax.dot_general(bblk, ones, (((0,), (0,)), ((), ())),
                        preferred_element_type=F32)

    @pl.when(i == 0)
    def _():
        acc[...] = part
        cnt[...] = c

    @pl.when(i > 0)
    def _():
        acc[...] += part
        cnt[...] += c

    @pl.when(i == nblk - 1)
    def _():
        hg = acc[...] / jnp.maximum(cnt[:, 0:1], 1.0)
        out_ref[...] = (jnp.dot(hg, pw_ref[...], preferred_element_type=F32)
                        + pb_ref[...])


def _final(z2, ss2, hl, b, pw, pb, n, emb, ncls):
    grid = (n // NB,)
    return pl.pallas_call(
        _final_body,
        grid=grid,
        in_specs=[
            pl.BlockSpec((NB, emb), lambda i: (i, 0)),
            pl.BlockSpec((2, emb), lambda i: (0, 0)),
            pl.BlockSpec((NB, emb), lambda i: (i, 0)),
            pl.BlockSpec((NB, 128), lambda i: (i, 0)),
            pl.BlockSpec((emb, ncls), lambda i: (0, 0)),
            pl.BlockSpec((1, ncls), lambda i: (0, 0)),
        ],
        out_specs=pl.BlockSpec((128, ncls), lambda i: (0, 0)),
        out_shape=jax.ShapeDtypeStruct((128, ncls), F32),
        scratch_shapes=[pltpu.VMEM((128, emb), F32), pltpu.VMEM((128, 128), F32)],
    )(z2, ss2, hl, b, pw, pb)


def _edge_aggr(y_flat, keys5, dst4, npad):
    keys2, dst = keys5, dst4
    e = dst.shape[0]
    ep = e // 16
    chunk = 128
    nfull = ep // chunk
    tail = ep - nfull * chunk
    pairs = nfull // 2
    assert nfull % 2 == 0 and ep % 8 == 0 and tail % 8 == 0 and tail <= 128
    rp = npad // 16
    zr = 32
    assert rp % zr == 0
    mesh = plsc.VectorSubcoreMesh(core_axis_name="c", subcore_axis_name="s",
                                  num_cores=2, num_subcores=16)

    @functools.partial(
        pl.kernel, mesh=mesh,
        out_type=jax.ShapeDtypeStruct((2 * npad, 128), F32),
        scratch_types=[
            pltpu.VMEM((chunk,), jnp.int32),
            pltpu.VMEM((chunk,), jnp.int32),
            pltpu.VMEM((chunk,), jnp.int32),
            pltpu.VMEM((chunk,), jnp.int32),
            pltpu.VMEM((tail,), jnp.int32),
            pltpu.VMEM((tail,), jnp.int32),
            pltpu.VMEM((chunk, 128), F32),
            pltpu.VMEM((chunk, 128), F32),
            pltpu.VMEM((tail, 128), F32),
            pltpu.VMEM((zr, 128), F32),
            pltpu.VMEM_SHARED((npad, 128), F32),
            pltpu.SemaphoreType.DMA,
            pltpu.SemaphoreType.DMA,
        ],
    )
    def k(y_hbm, keys_hbm, dst_hbm, out_hbm,
          kidx_a, didx_a, kidx_b, didx_b, kidx_t, didx_t,
          rows_a, rows_b, rows_t, zbuf, acc, sem_a, sem_b):
        c = lax.axis_index("c")
        s = lax.axis_index("s")

        def zrow(i, carry):
            for t in range(8):
                zbuf[i, pl.ds(t * 16, 16)] = jnp.zeros((16,), F32)
            return carry

        lax.fori_loop(0, zr, zrow, 0, unroll=False)
        for t in range(rp // zr):
            pltpu.sync_copy(zbuf, acc.at[pl.ds(s * rp + t * zr, zr)])

        ebase = s * ep
        kbase = c * e + ebase
        plsc.subcore_barrier()

        pltpu.sync_copy(keys_hbm.at[pl.ds(kbase, chunk)], kidx_a)
        pltpu.sync_copy(dst_hbm.at[pl.ds(ebase, chunk)], didx_a)
        pltpu.async_copy(y_hbm.at[kidx_a], rows_a, sem_a)

        def pair(t, carry):
            k0 = 2 * t * chunk
            pltpu.sync_copy(keys_hbm.at[pl.ds(kbase + k0 + chunk, chunk)], kidx_b)
            pltpu.sync_copy(dst_hbm.at[pl.ds(ebase + k0 + chunk, chunk)], didx_b)
            pltpu.async_copy(y_hbm.at[kidx_b], rows_b, sem_b)
            pltpu.make_async_copy(y_hbm.at[kidx_a], rows_a, sem_a).wait()
            pltpu.sync_copy(rows_a, acc.at[didx_a], add=True)

            @pl.when(t + 1 < pairs)
            def _():
                pltpu.sync_copy(keys_hbm.at[pl.ds(kbase + k0 + 2 * chunk, chunk)],
                                kidx_a)
                pltpu.sync_copy(dst_hbm.at[pl.ds(ebase + k0 + 2 * chunk, chunk)],
                                didx_a)
                pltpu.async_copy(y_hbm.at[kidx_a], rows_a, sem_a)

            pltpu.make_async_copy(y_hbm.at[kidx_b], rows_b, sem_b).wait()
            pltpu.sync_copy(rows_b, acc.at[didx_b], add=True)
            return carry

        lax.fori_loop(0, pairs, pair, 0)
        if tail:
            tbase = nfull * chunk
            pltpu.sync_copy(keys_hbm.at[pl.ds(kbase + tbase, tail)], kidx_t)
            pltpu.sync_copy(dst_hbm.at[pl.ds(ebase + tbase, tail)], didx_t)
            pltpu.async_copy(y_hbm.at[kidx_t], rows_t, sem_a).wait()
            pltpu.sync_copy(rows_t, acc.at[didx_t], add=True)
        plsc.subcore_barrier()
        pltpu.sync_copy(acc.at[pl.ds(s * rp, rp)],
                        out_hbm.at[pl.ds(c * npad + s * rp, rp)])

    return k(y_flat, keys2, dst)


def kernel(x, edge_index, edge_attr, batch, params):
    n = x.shape[0]
    e = edge_attr.shape[0]
    emb = params["atom_table"].shape[1]
    ncls = params["pred_W"].shape[1]
    nlayer = len(params["convs"])
    assert n % NB == 0 and emb == 256 and e % 16 == 0

    npad = ((n // 16 + 127) // 128) * 128 * 16

    x2 = x.astype(jnp.int32).reshape(n, 1)
    batch2 = batch.astype(jnp.int32).reshape(n, 1)
    vocab = params["atom_table"].shape[0]
    table_pad = jnp.concatenate(
        [params["atom_table"], jnp.zeros((128 - vocab, emb), F32)], axis=0)

    src = edge_index[0].astype(jnp.int32)
    dst = edge_index[1].astype(jnp.int32)
    keys = edge_attr.astype(jnp.int32) * n + src
    keys2 = jnp.concatenate([keys, keys + 5 * n])

    b_onehot, h0 = _prep(x2, batch2, table_pad, n)

    hl = h0
    z2 = ss2 = None
    vne = None
    for l in range(nlayer):
        p = params["convs"][l]
        bond = p["bond_table"]
        if l == 0:
            y = _ybuild(hl, bond, n, emb)
        else:
            hl, y = _t1b(z2, ss2, hl, b_onehot, vne, bond, n, emb)
        aggr = _edge_aggr(y.reshape(10 * n, 128), keys2, dst, npad)
        eps_row = (1.0 + p["eps"]) * jnp.ones((1, emb), F32)
        z1, ss1 = _p1(hl, aggr.reshape(2, npad, 128), eps_row, p["W1"],
                      p["bn1_g"].reshape(1, -1), p["bn1_b"].reshape(1, -1),
                      n, emb)
        z2, ss2 = _p2(z1, ss1, p["W2"],
                      p["bn_g"].reshape(1, -1), p["bn_b"].reshape(1, -1),
                      n, emb)
        if l < nlayer - 1:
            mp = params["vn_mlps"][l]
            vne_in = vne if vne is not None else jnp.zeros((128, emb), F32)
            vne = _vn(hl, b_onehot, vne_in, mp["Wa"],
                      mp["bn_a_g"].reshape(1, -1), mp["bn_a_b"].reshape(1, -1),
                      mp["Wb"],
                      mp["bn_b_g"].reshape(1, -1), mp["bn_b_b"].reshape(1, -1),
                      n, emb)

    return _final(z2, ss2, hl, b_onehot, params["pred_W"],
                  params["pred_b"].reshape(1, -1), n, emb, ncls)


# speedup vs baseline: 1.8680x; 1.2358x over previous
import functools

import jax
import jax.numpy as jnp
from jax import lax
from jax.experimental import pallas as pl
from jax.experimental.pallas import tpu as pltpu
from jax.experimental.pallas import tpu_sc as plsc

F32 = jnp.float32
NB = 1000


def _prep_body(x_ref, batch_ref, table_ref, b_ref, h0_ref):
    nb = x_ref.shape[0]
    iota_g = lax.broadcasted_iota(jnp.int32, (nb, 128), 1)
    b_ref[...] = (iota_g == batch_ref[...]).astype(F32)
    onehot_x = (iota_g == x_ref[...]).astype(F32)
    h0_ref[...] = jnp.dot(onehot_x, table_ref[...], preferred_element_type=F32)


def _prep(x2, batch2, table_pad, n):
    grid = (n // NB,)
    return pl.pallas_call(
        _prep_body,
        grid=grid,
        in_specs=[
            pl.BlockSpec((NB, 1), lambda i: (i, 0)),
            pl.BlockSpec((NB, 1), lambda i: (i, 0)),
            pl.BlockSpec(table_pad.shape, lambda i: (0, 0)),
        ],
        out_specs=[
            pl.BlockSpec((NB, 128), lambda i: (i, 0)),
            pl.BlockSpec((NB, table_pad.shape[1]), lambda i: (i, 0)),
        ],
        out_shape=[
            jax.ShapeDtypeStruct((n, 128), F32),
            jax.ShapeDtypeStruct((n, table_pad.shape[1]), F32),
        ],
    )(x2, batch2, table_pad)


def _write_y(y_ref, hl, bond_ref):
    for j in range(10):
        h, a = divmod(j, 5)
        sl = slice(h * 128, (h + 1) * 128)
        y_ref[j] = jnp.maximum(hl[:, sl] + bond_ref[a:a + 1, sl], 0.0)


def _ybuild_body(h_ref, bond_ref, y_ref):
    _write_y(y_ref, h_ref[...], bond_ref)


def _ybuild(h0, bond, n, emb):
    grid = (n // NB,)
    return pl.pallas_call(
        _ybuild_body,
        grid=grid,
        in_specs=[
            pl.BlockSpec((NB, emb), lambda i: (i, 0)),
            pl.BlockSpec((5, emb), lambda i: (0, 0)),
        ],
        out_specs=pl.BlockSpec((10, NB, 128), lambda i: (0, i, 0)),
        out_shape=jax.ShapeDtypeStruct((10, n, 128), F32),
    )(h0, bond)


def _t1b_body(z2_ref, ss_ref, hprev_ref, b_ref, vne_ref, bond_ref,
              hl_ref, y_ref):
    hnew = jnp.maximum(z2_ref[...] * ss_ref[0:1, :] + ss_ref[1:2, :], 0.0)
    hnew = hnew + hprev_ref[...]
    hl = hnew + jnp.dot(b_ref[...], vne_ref[...], preferred_element_type=F32)
    hl_ref[...] = hl
    _write_y(y_ref, hl, bond_ref)


def _t1b(z2, ss2, hprev, b, vne, bond, n, emb):
    grid = (n // NB,)
    return pl.pallas_call(
        _t1b_body,
        grid=grid,
        in_specs=[
            pl.BlockSpec((NB, emb), lambda i: (i, 0)),
            pl.BlockSpec((2, emb), lambda i: (0, 0)),
            pl.BlockSpec((NB, emb), lambda i: (i, 0)),
            pl.BlockSpec((NB, 128), lambda i: (i, 0)),
            pl.BlockSpec((128, emb), lambda i: (0, 0)),
            pl.BlockSpec((5, emb), lambda i: (0, 0)),
        ],
        out_specs=[
            pl.BlockSpec((NB, emb), lambda i: (i, 0)),
            pl.BlockSpec((10, NB, 128), lambda i: (0, i, 0)),
        ],
        out_shape=[
            jax.ShapeDtypeStruct((n, emb), F32),
            jax.ShapeDtypeStruct((10, n, 128), F32),
        ],
    )(z2, ss2, hprev, b, vne, bond)


def _mlp_stats_tail(i, nblk, n_rows, z, g_ref, bz_ref, ss_ref, acc_s, acc_q):
    s = jnp.sum(z, axis=0, keepdims=True)
    q = jnp.sum(z * z, axis=0, keepdims=True)

    @pl.when(i == 0)
    def _():
        acc_s[...] = s
        acc_q[...] = q

    @pl.when(i > 0)
    def _():
        acc_s[...] += s
        acc_q[...] += q

    @pl.when(i == nblk - 1)
    def _():
        mean = acc_s[...] * (1.0 / n_rows)
        var = acc_q[...] * (1.0 / n_rows) - mean * mean
        scale = lax.rsqrt(var + 1e-5) * g_ref[...]
        ss_ref[0:1, :] = scale
        ss_ref[1:2, :] = bz_ref[...] - mean * scale


def _p1_body(n_rows, hl_ref, agg_ref, eps_ref, w1_ref, g_ref, bz_ref,
             z1_ref, ss_ref, acc_s, acc_q):
    i = pl.program_id(0)
    nblk = pl.num_programs(0)
    aggr = jnp.concatenate([agg_ref[0], agg_ref[1]], axis=1)
    z = hl_ref[...] * eps_ref[...] + aggr
    z1 = jnp.dot(z, w1_ref[...], preferred_element_type=F32)
    z1_ref[...] = z1
    _mlp_stats_tail(i, nblk, n_rows, z1, g_ref, bz_ref, ss_ref, acc_s, acc_q)


def _p1(hl, aggr3, eps_row, w1, g1, bz1, n, emb):
    grid = (n // NB,)
    h = 2 * emb
    return pl.pallas_call(
        functools.partial(_p1_body, float(n)),
        grid=grid,
        in_specs=[
            pl.BlockSpec((NB, emb), lambda i: (i, 0)),
            pl.BlockSpec((2, NB, 128), lambda i: (0, i, 0)),
            pl.BlockSpec((1, emb), lambda i: (0, 0)),
            pl.BlockSpec((emb, h), lambda i: (0, 0)),
            pl.BlockSpec((1, h), lambda i: (0, 0)),
            pl.BlockSpec((1, h), lambda i: (0, 0)),
        ],
        out_specs=[
            pl.BlockSpec((NB, h), lambda i: (i, 0)),
            pl.BlockSpec((2, h), lambda i: (0, 0)),
        ],
        out_shape=[
            jax.ShapeDtypeStruct((n, h), F32),
            jax.ShapeDtypeStruct((2, h), F32),
        ],
        scratch_shapes=[pltpu.VMEM((1, h), F32), pltpu.VMEM((1, h), F32)],
    )(hl, aggr3, eps_row, w1, g1, bz1)


def _p2_body(n_rows, z1_ref, ss1_ref, w2_ref, g_ref, bz_ref,
             z2_ref, ss2_ref, acc_s, acc_q):
    i = pl.program_id(0)
    nblk = pl.num_programs(0)
    a = jnp.maximum(z1_ref[...] * ss1_ref[0:1, :] + ss1_ref[1:2, :], 0.0)
    z2 = jnp.dot(a, w2_ref[...], preferred_element_type=F32)
    z2_ref[...] = z2
    _mlp_stats_tail(i, nblk, n_rows, z2, g_ref, bz_ref, ss2_ref, acc_s, acc_q)


def _p2(z1, ss1, w2, g2, bz2, n, emb):
    grid = (n // NB,)
    h = 2 * emb
    return pl.pallas_call(
        functools.partial(_p2_body, float(n)),
        grid=grid,
        in_specs=[
            pl.BlockSpec((NB, h), lambda i: (i, 0)),
            pl.BlockSpec((2, h), lambda i: (0, 0)),
            pl.BlockSpec((h, emb), lambda i: (0, 0)),
            pl.BlockSpec((1, emb), lambda i: (0, 0)),
            pl.BlockSpec((1, emb), lambda i: (0, 0)),
        ],
        out_specs=[
            pl.BlockSpec((NB, emb), lambda i: (i, 0)),
            pl.BlockSpec((2, emb), lambda i: (0, 0)),
        ],
        out_shape=[
            jax.ShapeDtypeStruct((n, emb), F32),
            jax.ShapeDtypeStruct((2, emb), F32),
        ],
        scratch_shapes=[pltpu.VMEM((1, emb), F32), pltpu.VMEM((1, emb), F32)],
    )(z1, ss1, w2, g2, bz2)


def _bn_rows(t, g, b):
    mean = jnp.mean(t, axis=0, keepdims=True)
    var = jnp.mean(jnp.square(t - mean), axis=0, keepdims=True)
    return (t - mean) * lax.rsqrt(var + 1e-5) * g + b


def _vn_body(hl_ref, b_ref, vne_ref, wa_ref, ga_ref, bza_ref,
             wb_ref, gb_ref, bzb_ref, vout_ref, acc):
    i = pl.program_id(0)
    nblk = pl.num_programs(0)
    part = lax.dot_general(b_ref[...], hl_ref[...],
                           (((0,), (0,)), ((), ())),
                           preferred_element_type=F32)

    @pl.when(i == 0)
    def _():
        acc[...] = part

    @pl.when(i > 0)
    def _():
        acc[...] += part

    @pl.when(i == nblk - 1)
    def _():
        tmp = acc[...] + vne_ref[...]
        t = jnp.dot(tmp, wa_ref[...], preferred_element_type=F32)
        t = jnp.maximum(_bn_rows(t, ga_ref[...], bza_ref[...]), 0.0)
        t = jnp.dot(t, wb_ref[...], preferred_element_type=F32)
        t = jnp.maximum(_bn_rows(t, gb_ref[...], bzb_ref[...]), 0.0)
        vout_ref[...] = vne_ref[...] + t


def _vn(hl, b, vne, wa, ga, bza, wb, gb, bzb, n, emb):
    grid = (n // NB,)
    h = 2 * emb
    return pl.pallas_call(
        _vn_body,
        grid=grid,
        in_specs=[
            pl.BlockSpec((NB, emb), lambda i: (i, 0)),
            pl.BlockSpec((NB, 128), lambda i: (i, 0)),
            pl.BlockSpec((128, emb), lambda i: (0, 0)),
            pl.BlockSpec((emb, h), lambda i: (0, 0)),
            pl.BlockSpec((1, h), lambda i: (0, 0)),
            pl.BlockSpec((1, h), lambda i: (0, 0)),
            pl.BlockSpec((h, emb), lambda i: (0, 0)),
            pl.BlockSpec((1, emb), lambda i: (0, 0)),
            pl.BlockSpec((1, emb), lambda i: (0, 0)),
        ],
        out_specs=pl.BlockSpec((128, emb), lambda i: (0, 0)),
        out_shape=jax.ShapeDtypeStruct((128, emb), F32),
        scratch_shapes=[pltpu.VMEM((128, emb), F32)],
    )(hl, b, vne, wa, ga, bza, wb, gb, bzb)


def _final_body(z2_ref, ss2_ref, hl_ref, b_ref, pw_ref, pb_ref,
                out_ref, acc, cnt):
    i = pl.program_id(0)
    nblk = pl.num_programs(0)
    hnode = z2_ref[...] * ss2_ref[0:1, :] + ss2_ref[1:2, :] + hl_ref[...]
    bblk = b_ref[...]
    part = lax.dot_general(bblk, hnode, (((0,), (0,)), ((), ())),
                           preferred_element_type=F32)
    ones = jnp.ones(bblk.shape[:1] + (128,), F32)
    c = lax.dot_general(bblk, ones, (((0,), (0,)), ((), ())),
                        preferred_element_type=F32)

    @pl.when(i == 0)
    def _():
        acc[...] = part
        cnt[...] = c

    @pl.when(i > 0)
    def _():
        acc[...] += part
        cnt[...] += c

    @pl.when(i == nblk - 1)
    def _():
        hg = acc[...] / jnp.maximum(cnt[:, 0:1], 1.0)
        out_ref[...] = (jnp.dot(hg, pw_ref[...], preferred_element_type=F32)
                        + pb_ref[...])


def _final(z2, ss2, hl, b, pw, pb, n, emb, ncls):
    grid = (n // NB,)
    return pl.pallas_call(
        _final_body,
        grid=grid,
        in_specs=[
            pl.BlockSpec((NB, emb), lambda i: (i, 0)),
            pl.BlockSpec((2, emb), lambda i: (0, 0)),
            pl.BlockSpec((NB, emb), lambda i: (i, 0)),
            pl.BlockSpec((NB, 128), lambda i: (i, 0)),
            pl.BlockSpec((emb, ncls), lambda i: (0, 0)),
            pl.BlockSpec((1, ncls), lambda i: (0, 0)),
        ],
        out_specs=pl.BlockSpec((128, ncls), lambda i: (0, 0)),
        out_shape=jax.ShapeDtypeStruct((128, ncls), F32),
        scratch_shapes=[pltpu.VMEM((128, emb), F32), pltpu.VMEM((128, 128), F32)],
    )(z2, ss2, hl, b, pw, pb)


def _edge_aggr(y_flat, keys5, dst4, npad):
    keys2, dst = keys5, dst4
    e = dst.shape[0]
    ep = e // 16
    chunk = 128
    nfull = ep // chunk
    tail = ep - nfull * chunk
    assert nfull % 6 == 0 and ep % 8 == 0 and tail % 8 == 0 and tail <= 128
    rp = npad // 16
    zr = 32
    assert rp % zr == 0
    mesh = plsc.VectorSubcoreMesh(core_axis_name="c", subcore_axis_name="s",
                                  num_cores=2, num_subcores=16)

    @functools.partial(
        pl.kernel, mesh=mesh,
        out_type=jax.ShapeDtypeStruct((2 * npad, 128), F32),
        scratch_types=[
            pltpu.VMEM((chunk,), jnp.int32),
            pltpu.VMEM((chunk,), jnp.int32),
            pltpu.VMEM((chunk,), jnp.int32),
            pltpu.VMEM((chunk,), jnp.int32),
            pltpu.VMEM((chunk,), jnp.int32),
            pltpu.VMEM((chunk,), jnp.int32),
            pltpu.VMEM((tail,), jnp.int32),
            pltpu.VMEM((tail,), jnp.int32),
            pltpu.VMEM((chunk, 128), F32),
            pltpu.VMEM((chunk, 128), F32),
            pltpu.VMEM((tail, 128), F32),
            pltpu.VMEM((zr, 128), F32),
            pltpu.VMEM_SHARED((npad, 128), F32),
            pltpu.SemaphoreType.DMA,
            pltpu.SemaphoreType.DMA,
            pltpu.SemaphoreType.DMA,
            pltpu.SemaphoreType.DMA,
            pltpu.SemaphoreType.DMA,
        ],
    )
    def k(y_hbm, keys_hbm, dst_hbm, out_hbm,
          k0b, d0b, k1b, d1b, k2b, d2b, kidx_t, didx_t,
          rows_a, rows_b, rows_t, zbuf, acc,
          sem_a, sem_b, si0, si1, si2):
        kidx = [k0b, k1b, k2b]
        didx = [d0b, d1b, d2b]
        rows = [rows_a, rows_b]
        sem_g = [sem_a, sem_b]
        sem_i = [si0, si1, si2]
        c = lax.axis_index("c")
        s = lax.axis_index("s")

        def zrow(i, carry):
            for t in range(8):
                zbuf[i, pl.ds(t * 16, 16)] = jnp.zeros((16,), F32)
            return carry

        lax.fori_loop(0, zr, zrow, 0, unroll=False)
        for t in range(rp // zr):
            pltpu.sync_copy(zbuf, acc.at[pl.ds(s * rp + t * zr, zr)])

        ebase = s * ep
        kbase = c * e + ebase

        def fire_idx(kk, m):
            pltpu.async_copy(keys_hbm.at[pl.ds(kbase + kk * chunk, chunk)],
                             kidx[m], sem_i[m])
            pltpu.async_copy(dst_hbm.at[pl.ds(ebase + kk * chunk, chunk)],
                             didx[m], sem_i[m])

        def wait_idx(kk, m):
            pltpu.make_async_copy(keys_hbm.at[pl.ds(kbase + kk * chunk, chunk)],
                                  kidx[m], sem_i[m]).wait()
            pltpu.make_async_copy(dst_hbm.at[pl.ds(ebase + kk * chunk, chunk)],
                                  didx[m], sem_i[m]).wait()

        pltpu.sync_copy(keys_hbm.at[pl.ds(kbase, chunk)], kidx[0])
        pltpu.sync_copy(dst_hbm.at[pl.ds(ebase, chunk)], didx[0])
        fire_idx(1, 1)
        plsc.subcore_barrier()
        pltpu.async_copy(y_hbm.at[kidx[0]], rows[0], sem_g[0])

        def six(i, carry):
            for j in range(6):
                kk = 6 * i + j
                m = j % 3
                mn = (j + 1) % 3
                mf = (j + 2) % 3
                rp_ = j % 2
                rn = (j + 1) % 2

                @pl.when(kk + 2 < nfull)
                def _():
                    fire_idx(kk + 2, mf)

                @pl.when(kk + 1 < nfull)
                def _():
                    wait_idx(kk + 1, mn)
                    pltpu.async_copy(y_hbm.at[kidx[mn]], rows[rn], sem_g[rn])

                pltpu.make_async_copy(y_hbm.at[kidx[m]], rows[rp_],
                                      sem_g[rp_]).wait()
                pltpu.sync_copy(rows[rp_], acc.at[didx[m]], add=True)
            return carry

        lax.fori_loop(0, nfull // 6, six, 0)
        if tail:
            tbase = nfull * chunk
            pltpu.sync_copy(keys_hbm.at[pl.ds(kbase + tbase, tail)], kidx_t)
            pltpu.sync_copy(dst_hbm.at[pl.ds(ebase + tbase, tail)], didx_t)
            pltpu.async_copy(y_hbm.at[kidx_t], rows_t, sem_a).wait()
            pltpu.sync_copy(rows_t, acc.at[didx_t], add=True)
        plsc.subcore_barrier()
        pltpu.sync_copy(acc.at[pl.ds(s * rp, rp)],
                        out_hbm.at[pl.ds(c * npad + s * rp, rp)])

    return k(y_flat, keys2, dst)


def kernel(x, edge_index, edge_attr, batch, params):
    n = x.shape[0]
    e = edge_attr.shape[0]
    emb = params["atom_table"].shape[1]
    ncls = params["pred_W"].shape[1]
    nlayer = len(params["convs"])
    assert n % NB == 0 and emb == 256 and e % 16 == 0

    npad = ((n // 16 + 127) // 128) * 128 * 16

    x2 = x.astype(jnp.int32).reshape(n, 1)
    batch2 = batch.astype(jnp.int32).reshape(n, 1)
    vocab = params["atom_table"].shape[0]
    table_pad = jnp.concatenate(
        [params["atom_table"], jnp.zeros((128 - vocab, emb), F32)], axis=0)

    src = edge_index[0].astype(jnp.int32)
    dst = edge_index[1].astype(jnp.int32)
    keys = edge_attr.astype(jnp.int32) * n + src
    keys2 = jnp.concatenate([keys, keys + 5 * n])

    b_onehot, h0 = _prep(x2, batch2, table_pad, n)

    hl = h0
    z2 = ss2 = None
    vne = None
    for l in range(nlayer):
        p = params["convs"][l]
        bond = p["bond_table"]
        if l == 0:
            y = _ybuild(hl, bond, n, emb)
        else:
            hl, y = _t1b(z2, ss2, hl, b_onehot, vne, bond, n, emb)
        aggr = _edge_aggr(y.reshape(10 * n, 128), keys2, dst, npad)
        eps_row = (1.0 + p["eps"]) * jnp.ones((1, emb), F32)
        z1, ss1 = _p1(hl, aggr.reshape(2, npad, 128), eps_row, p["W1"],
                      p["bn1_g"].reshape(1, -1), p["bn1_b"].reshape(1, -1),
                      n, emb)
        z2, ss2 = _p2(z1, ss1, p["W2"],
                      p["bn_g"].reshape(1, -1), p["bn_b"].reshape(1, -1),
                      n, emb)
        if l < nlayer - 1:
            mp = params["vn_mlps"][l]
            vne_in = vne if vne is not None else jnp.zeros((128, emb), F32)
            vne = _vn(hl, b_onehot, vne_in, mp["Wa"],
                      mp["bn_a_g"].reshape(1, -1), mp["bn_a_b"].reshape(1, -1),
                      mp["Wb"],
                      mp["bn_b_g"].reshape(1, -1), mp["bn_b_b"].reshape(1, -1),
                      n, emb)

    return _final(z2, ss2, hl, b_onehot, params["pred_W"],
                  params["pred_b"].reshape(1, -1), n, emb, ncls)
